# Initial kernel scaffold; baseline (speedup 1.0000x reference)
import jax
import jax.numpy as jnp
from jax.experimental import pallas as pl


def kernel(features, edge_index, W1, b1, W2, b2):
    raise NotImplementedError("write your pallas kernel here")


# speedup vs baseline: 4.6757x; 4.6757x over previous
import functools

import jax
import jax.numpy as jnp
from jax import lax
from jax.experimental import pallas as pl
from jax.experimental.pallas import tpu as pltpu
from jax.experimental.pallas import tpu_sc as plsc

N_NODES = 10000
N_EDGES = 160000
IN_DIM = 256
HIDDEN = 512
OUT_DIM = 256

DH = 128
NS = 16
E_PER_TILE = N_EDGES // NS
K = 80
N_STEPS = E_PER_TILE // K
ROWS_PER_TILE = 624
TAIL_OFF = ROWS_PER_TILE * NS
TAIL = N_NODES - TAIL_OFF


def _seg_kernel(x0_hbm, x1_hbm, src_hbm, dst_hbm, o0_hbm, o1_hbm,
                sidx, didx, rows, acc, sem):
    cid = lax.axis_index("c")
    sid = lax.axis_index("s")

    def run(x_hbm, o_hbm):
        r0 = sid * ROWS_PER_TILE
        pltpu.sync_copy(x_hbm.at[pl.ds(r0, ROWS_PER_TILE)],
                        acc.at[pl.ds(r0, ROWS_PER_TILE)])

        @pl.when(sid == NS - 1)
        def _():
            pltpu.sync_copy(x_hbm.at[pl.ds(TAIL_OFF, TAIL)],
                            acc.at[pl.ds(TAIL_OFF, TAIL)])

        plsc.subcore_barrier()

        e0 = sid * E_PER_TILE

        def step(i, carry):
            off = e0 + i * K
            pltpu.sync_copy(src_hbm.at[pl.ds(off, K)], sidx)
            pltpu.sync_copy(dst_hbm.at[pl.ds(off, K)], didx)
            pltpu.async_copy(x_hbm.at[sidx], rows, sem).wait()
            pltpu.sync_copy(rows, acc.at[didx], add=True)
            return carry

        lax.fori_loop(0, N_STEPS, step, 0)
        plsc.subcore_barrier()
        pltpu.sync_copy(acc.at[pl.ds(r0, ROWS_PER_TILE)],
                        o_hbm.at[pl.ds(r0, ROWS_PER_TILE)])

        @pl.when(sid == NS - 1)
        def _():
            pltpu.sync_copy(acc.at[pl.ds(TAIL_OFF, TAIL)],
                            o_hbm.at[pl.ds(TAIL_OFF, TAIL)])

    @pl.when(cid == 0)
    def _():
        run(x0_hbm, o0_hbm)

    @pl.when(cid == 1)
    def _():
        run(x1_hbm, o1_hbm)


_seg_sum = functools.partial(
    pl.kernel,
    _seg_kernel,
    out_type=(jax.ShapeDtypeStruct((N_NODES, DH), jnp.float32),
              jax.ShapeDtypeStruct((N_NODES, DH), jnp.float32)),
    mesh=plsc.VectorSubcoreMesh(core_axis_name="c", subcore_axis_name="s"),
    scratch_types=[
        pltpu.VMEM((K,), jnp.int32),
        pltpu.VMEM((K,), jnp.int32),
        pltpu.VMEM((K, DH), jnp.float32),
        pltpu.VMEM_SHARED((N_NODES, DH), jnp.float32),
        pltpu.SemaphoreType.DMA,
    ],
    name="gin_seg_sum",
)()

MB = 1000


def _mm_body(h0_ref, h1_ref, w1_ref, b1_ref, w2_ref, y0_ref, y1_ref):
    h = jnp.concatenate([h0_ref[...], h1_ref[...]], axis=1)
    x2 = jnp.maximum(jnp.dot(h, w1_ref[...],
                             preferred_element_type=jnp.float32) + b1_ref[...], 0.0)
    y2 = jnp.dot(x2, w2_ref[...], preferred_element_type=jnp.float32)
    y0_ref[...] = y2[:, :DH]
    y1_ref[...] = y2[:, DH:]


_mm = pl.pallas_call(
    _mm_body,
    grid=(N_NODES // MB,),
    in_specs=[
        pl.BlockSpec((MB, DH), lambda i: (i, 0)),
        pl.BlockSpec((MB, DH), lambda i: (i, 0)),
        pl.BlockSpec((IN_DIM, HIDDEN), lambda i: (0, 0)),
        pl.BlockSpec((1, HIDDEN), lambda i: (0, 0)),
        pl.BlockSpec((HIDDEN, OUT_DIM), lambda i: (0, 0)),
    ],
    out_specs=[
        pl.BlockSpec((MB, DH), lambda i: (i, 0)),
        pl.BlockSpec((MB, DH), lambda i: (i, 0)),
    ],
    out_shape=[
        jax.ShapeDtypeStruct((N_NODES, DH), jnp.float32),
        jax.ShapeDtypeStruct((N_NODES, DH), jnp.float32),
    ],
)


def _softmax_body(h0_ref, h1_ref, b2_ref, o_ref):
    z = jnp.concatenate([h0_ref[...], h1_ref[...]], axis=1) + b2_ref[...]
    m = jnp.max(z, axis=1, keepdims=True)
    e = jnp.exp(z - m)
    s = jnp.sum(e, axis=1, keepdims=True)
    o_ref[...] = (z - m) - jnp.log(s)


_softmax = pl.pallas_call(
    _softmax_body,
    grid=(N_NODES // MB,),
    in_specs=[
        pl.BlockSpec((MB, DH), lambda i: (i, 0)),
        pl.BlockSpec((MB, DH), lambda i: (i, 0)),
        pl.BlockSpec((1, OUT_DIM), lambda i: (0, 0)),
    ],
    out_specs=pl.BlockSpec((MB, OUT_DIM), lambda i: (i, 0)),
    out_shape=jax.ShapeDtypeStruct((N_NODES, OUT_DIM), jnp.float32),
)


def kernel(features, edge_index, W1, b1, W2, b2):
    src = edge_index[0].astype(jnp.int32)
    dst = edge_index[1].astype(jnp.int32)
    x0 = features[:, :DH]
    x1 = features[:, DH:]
    hp10, hp11 = _seg_sum(x0, x1, src, dst)
    y0, y1 = _mm(hp10, hp11, W1, b1.reshape(1, HIDDEN), W2)
    hp20, hp21 = _seg_sum(y0, y1, src, dst)
    return _softmax(hp20, hp21, b2.reshape(1, OUT_DIM))
